# baseline (speedup 1.0000x reference)
import functools

import jax
import jax.numpy as jnp
import numpy as np
from jax import lax
from jax.experimental import pallas as pl
from jax.experimental.pallas import tpu as pltpu
from jax.experimental.pallas import tpu_sc as plsc

N = 10000
NP = 10240
E = 320000
G = 64
H = 128
SENT = 10000
BN_EPS = 1e-5
_BN_SCALE = float(1.0 / np.sqrt(1.0 + BN_EPS))

NC, NS = 2, 16
NW = NC * NS
ECH = 40
EPW = E // NW
NCHK = EPW // ECH
RPW = NP // NW
RPT = NP // NS

@functools.cache
def _sc_mesh():
    return plsc.VectorSubcoreMesh(
        core_axis_name="c", subcore_axis_name="s",
        num_cores=NC, num_subcores=NS)

f32 = jnp.float32
i32 = jnp.int32


_NBUF = 8
_NGRP = NCHK // _NBUF
_NTAIL = NCHK - _NGRP * _NBUF


def _agg_body(x_hbm, src_hbm, dst_hbm, zz_hbm, out_hbm, *refs):
    src2 = refs[0]
    dst2 = refs[1]
    rows_v = refs[2:2 + _NBUF]
    acc_sh = refs[2 + _NBUF]
    sems = refs[3 + _NBUF:]
    cid = lax.axis_index("c")
    sid = lax.axis_index("s")
    pltpu.sync_copy(zz_hbm, acc_sh.at[pl.ds(sid * RPT, RPT)])
    plsc.subcore_barrier()

    wid = cid * NS + sid

    def start(b):
        return pltpu.async_copy(x_hbm.at[src2.at[b]], rows_v[b], sems[b])

    def finish(d, b):
        d.wait()
        pltpu.sync_copy(rows_v[b], acc_sh.at[dst2.at[b]], add=True)

    @pl.loop(0, _NGRP)
    def _grp(g):
        pltpu.sync_copy(src_hbm.at[wid, pl.ds(g * _NBUF, _NBUF)], src2)
        pltpu.sync_copy(dst_hbm.at[wid, pl.ds(g * _NBUF, _NBUF)], dst2)
        ds = [start(b) for b in range(_NBUF)]
        for b in range(_NBUF):
            finish(ds[b], b)

    pltpu.sync_copy(src_hbm.at[wid, pl.ds(_NGRP * _NBUF, _NTAIL)],
                    src2.at[pl.ds(0, _NTAIL)])
    pltpu.sync_copy(dst_hbm.at[wid, pl.ds(_NGRP * _NBUF, _NTAIL)],
                    dst2.at[pl.ds(0, _NTAIL)])
    tail = [start(b) for b in range(_NTAIL)]
    for b in range(_NTAIL):
        finish(tail[b], b)

    plsc.subcore_barrier()
    pltpu.sync_copy(acc_sh.at[pl.ds(sid * RPT, RPT)],
                    out_hbm.at[cid, pl.ds(sid * RPT, RPT)])


@functools.cache
def _agg_kernel():
    return pl.kernel(
        _agg_body,
        out_type=jax.ShapeDtypeStruct((NC, NP, H), f32),
        mesh=_sc_mesh(),
        scratch_types=(
            [pltpu.VMEM((_NBUF, ECH), i32) for _ in range(2)]
            + [pltpu.VMEM((ECH, H), f32) for _ in range(_NBUF)]
            + [pltpu.VMEM_SHARED((NP, H), f32)]
            + [pltpu.SemaphoreType.DMA for _ in range(_NBUF)]
        ),
    )


def _agg(x_p, src, dst, zz):
    return _agg_kernel()(x_p, src.reshape(NW, NCHK, ECH),
                         dst.reshape(NW, NCHK, ECH), zz)


_ECH2 = 400


def _pool_body(h1s_hbm, np_hbm, src_hbm, dst_hbm,
               xnew_hbm, ns_hbm, nd_hbm,
               npos_v, rows_v, npc_v, sidx_v, didx_v, nsb_v, ndb_v):
    cid = lax.axis_index("c")
    sid = lax.axis_index("s")
    wid = cid * NS + sid

    nb = wid * RPW

    @pl.loop(0, RPW // ECH)
    def _nchunk(c):
        pltpu.sync_copy(h1s_hbm.at[pl.ds(nb + c * ECH, ECH)], rows_v)
        pltpu.sync_copy(np_hbm.at[pl.ds(nb + c * ECH, ECH)], npc_v)
        pltpu.sync_copy(rows_v, xnew_hbm.at[npc_v])

    pltpu.sync_copy(np_hbm, npos_v)
    eb = wid * EPW

    @pl.loop(0, EPW // _ECH2)
    def _echunk(c):
        pltpu.sync_copy(src_hbm.at[pl.ds(eb + c * _ECH2, _ECH2)], sidx_v)
        pltpu.sync_copy(dst_hbm.at[pl.ds(eb + c * _ECH2, _ECH2)], didx_v)

        @pl.loop(0, _ECH2 // 16)
        def _vec(k):
            s16 = sidx_v[pl.ds(k * 16, 16)]
            d16 = didx_v[pl.ds(k * 16, 16)]
            ps = plsc.load_gather(npos_v, [s16])
            pd = plsc.load_gather(npos_v, [d16])
            valid = (ps < SENT) & (pd < SENT)
            nsb_v[pl.ds(k * 16, 16)] = jnp.where(valid, ps, SENT + (s16 & 127))
            ndb_v[pl.ds(k * 16, 16)] = jnp.where(valid, pd, SENT + (d16 & 127))

        pltpu.sync_copy(nsb_v, ns_hbm.at[pl.ds(eb + c * _ECH2, _ECH2)])
        pltpu.sync_copy(ndb_v, nd_hbm.at[pl.ds(eb + c * _ECH2, _ECH2)])


@functools.cache
def _pool_kernel():
    return pl.kernel(
        _pool_body,
        out_type=(
            jax.ShapeDtypeStruct((NP, H), f32),
            jax.ShapeDtypeStruct((E,), i32),
            jax.ShapeDtypeStruct((E,), i32),
        ),
        mesh=_sc_mesh(),
        compiler_params=pltpu.CompilerParams(needs_layout_passes=False),
        scratch_types=[
            pltpu.VMEM((NP,), i32),
            pltpu.VMEM((ECH, H), f32),
            pltpu.VMEM((ECH,), i32),
            pltpu.VMEM((_ECH2,), i32),
            pltpu.VMEM((_ECH2,), i32),
            pltpu.VMEM((_ECH2,), i32),
            pltpu.VMEM((_ECH2,), i32),
        ],
    )


def _pool(h1s, npos, src, dst):
    return _pool_kernel()(h1s, npos, src, dst)


def _counts_body(b_ref, out_ref):
    i = pl.program_id(0)

    @pl.when(i == 0)
    def _():
        out_ref[...] = jnp.zeros_like(out_ref)

    b = b_ref[0]
    gcol = lax.broadcasted_iota(i32, (G, 128), 0).astype(f32)
    eq = (jnp.broadcast_to(b, (G, 128)) == gcol).astype(f32)
    out_ref[...] += jnp.sum(eq, axis=1, keepdims=True)


def _counts(batch_f):
    return pl.pallas_call(
        _counts_body,
        grid=(NP // 128,),
        in_specs=[pl.BlockSpec((1, 1, 128), lambda i: (i, 0, 0))],
        out_specs=pl.BlockSpec((G, 1), lambda i: (0, 0)),
        out_shape=jax.ShapeDtypeStruct((G, 1), f32),
    )(batch_f)


R = 1024


def _mlp_body(lo_ref, hi_ref, x_ref, p0_ref, p1_ref, w1_ref, w2_ref, c_ref,
              h_ref, xs_ref):
    i = pl.program_id(0)
    t = x_ref[...] + p0_ref[...] + p1_ref[...]
    a = jnp.dot(t, w1_ref[...], preferred_element_type=f32) + c_ref[0:1, :]
    a = jnp.maximum(a, 0.0)
    a = jnp.dot(a, w2_ref[...], preferred_element_type=f32) + c_ref[1:2, :]
    a = jnp.maximum(a, 0.0)
    h = jnp.maximum(a * (c_ref[2:3, :] * _BN_SCALE) + c_ref[3:4, :], 0.0)
    h_ref[...] = h

    p = lax.broadcasted_iota(i32, (R, G), 0).astype(f32) + (i * R).astype(f32)
    oh = ((p >= lo_ref[...]) & (p < hi_ref[...])).astype(f32)
    vmask = jnp.sum(oh, axis=1, keepdims=True) > 0.0
    hm = jnp.where(vmask, h, 0.0)
    contrib = lax.dot_general(oh, hm, (((0,), (0,)), ((), ())),
                              preferred_element_type=f32)

    @pl.when(i == 0)
    def _():
        xs_ref[...] = jnp.zeros_like(xs_ref)

    xs_ref[...] += contrib


def _mlp(xin, p0, p1, w1, b1, w2, b2, g, be, lo, hi):
    cvec = jnp.stack([b1, b2, g, be], axis=0)
    return pl.pallas_call(
        _mlp_body,
        grid=(NP // R,),
        in_specs=[
            pl.BlockSpec((1, G), lambda i: (0, 0)),
            pl.BlockSpec((1, G), lambda i: (0, 0)),
            pl.BlockSpec((R, H), lambda i: (i, 0)),
            pl.BlockSpec((R, H), lambda i: (i, 0)),
            pl.BlockSpec((R, H), lambda i: (i, 0)),
            pl.BlockSpec((H, H), lambda i: (0, 0)),
            pl.BlockSpec((H, H), lambda i: (0, 0)),
            pl.BlockSpec((4, H), lambda i: (0, 0)),
        ],
        out_specs=[
            pl.BlockSpec((R, H), lambda i: (i, 0)),
            pl.BlockSpec((G, H), lambda i: (0, 0)),
        ],
        out_shape=[
            jax.ShapeDtypeStruct((NP, H), f32),
            jax.ShapeDtypeStruct((G, H), f32),
        ],
    )(lo, hi, xin, p0, p1, w1, w2, cvec)


def _rank_body(batch_s, ptr_s, h_ref, w_ref, loc_ref, hic_ref, lor_ref,
               hir_ref, kt_ref, pnt_ref, np_ref, h1s_ref):
    ic = pl.program_id(0)
    w = w_ref[...]
    nrm = jnp.sqrt(jnp.sum(w * w)) + 1e-16

    hi_blk = h_ref[pl.ds(ic * 128, 128), :]
    s_col = jnp.tanh(jnp.dot(hi_blk, w, preferred_element_type=f32) / nrm)
    h1s_ref[...] = hi_blk * s_col
    eyeb = (lax.broadcasted_iota(i32, (128, 128), 0)
            == lax.broadcasted_iota(i32, (128, 128), 1))
    s_row = jnp.max(jnp.where(eyeb, jnp.broadcast_to(s_col, (128, 128)),
                              -jnp.inf), axis=0, keepdims=True)

    p_i = lax.broadcasted_iota(i32, (G, 128), 1).astype(f32) + (ic * 128).astype(f32)
    ohi = ((p_i >= loc_ref[...]) & (p_i < hic_ref[...])).astype(f32)
    gcol = lax.broadcasted_iota(i32, (G, 128), 0).astype(f32)
    anyi = jnp.sum(ohi, axis=0, keepdims=True)
    bi_row = jnp.sum(ohi * gcol, axis=0, keepdims=True) + (1.0 - anyi) * 64.0
    k_i = jnp.sum(ohi * kt_ref[...], axis=0, keepdims=True)
    pn_i = jnp.sum(ohi * pnt_ref[...], axis=0, keepdims=True)

    idx_i = lax.broadcasted_iota(i32, (128, 128), 1) + ic * 128

    glo = jnp.minimum(batch_s[ic * 128], 63)
    ghi = jnp.minimum(batch_s[jnp.minimum(ic * 128 + 127, N - 1)], 63)
    jlo = ptr_s[glo] // 128
    jhi = (ptr_s[ghi + 1] + 127) // 128

    def jbody(jc, cnt):
        hj = h_ref[pl.ds(jc * 128, 128), :]
        sj = jnp.tanh(jnp.dot(hj, w, preferred_element_type=f32) / nrm)
        p_j = lax.broadcasted_iota(i32, (128, G), 0).astype(f32) + (jc * 128).astype(f32)
        ohj = ((p_j >= lor_ref[...]) & (p_j < hir_ref[...])).astype(f32)
        grow = lax.broadcasted_iota(i32, (128, G), 1).astype(f32)
        anyj = jnp.sum(ohj, axis=1, keepdims=True)
        bj = jnp.sum(ohj * grow, axis=1, keepdims=True) + (1.0 - anyj) * 64.0
        idx_j = lax.broadcasted_iota(i32, (128, 128), 0) + jc * 128
        m = (bj == bi_row) & ((sj > s_row) | ((sj == s_row) & (idx_j < idx_i)))
        return cnt + jnp.sum(m.astype(f32), axis=0, keepdims=True)

    cnt = lax.fori_loop(jlo, jhi, jbody, jnp.zeros((1, 128), f32))

    kept = cnt < k_i
    lane = lax.broadcasted_iota(i32, (1, 128), 1).astype(f32)
    np_ref[0] = jnp.where(kept, pn_i + cnt, jnp.float32(SENT) + lane)


def _rank(batch_p, ptr, h1, w, loc, hic, lor, hir, ktab, pntab):
    grid_spec = pltpu.PrefetchScalarGridSpec(
        num_scalar_prefetch=2,
        grid=(NP // 128,),
        in_specs=[
            pl.BlockSpec((NP, H), lambda ic, *_: (0, 0)),
            pl.BlockSpec((H, 1), lambda ic, *_: (0, 0)),
            pl.BlockSpec((G, 1), lambda ic, *_: (0, 0)),
            pl.BlockSpec((G, 1), lambda ic, *_: (0, 0)),
            pl.BlockSpec((1, G), lambda ic, *_: (0, 0)),
            pl.BlockSpec((1, G), lambda ic, *_: (0, 0)),
            pl.BlockSpec((G, 1), lambda ic, *_: (0, 0)),
            pl.BlockSpec((G, 1), lambda ic, *_: (0, 0)),
        ],
        out_specs=[
            pl.BlockSpec((1, 1, 128), lambda ic, *_: (ic, 0, 0)),
            pl.BlockSpec((128, H), lambda ic, *_: (ic, 0)),
        ],
    )
    return pl.pallas_call(
        _rank_body,
        grid_spec=grid_spec,
        out_shape=[
            jax.ShapeDtypeStruct((NP // 128, 1, 128), f32),
            jax.ShapeDtypeStruct((NP, H), f32),
        ],
    )(batch_p, ptr, h1, w, loc, hic, lor, hir, ktab, pntab)


def _head_body(xs0, xs1, xs2, xs3, c0, c1, wl1, bl1, wf1, bf1, g1, be1,
               wf2, bf2, g2, be2, wl2, bl2, out_ref):
    ic0 = 1.0 / jnp.maximum(c0[...], 1.0)
    ic1 = 1.0 / jnp.maximum(c1[...], 1.0)
    z = (jnp.dot(xs0[...] * ic0, wl1[0:H, :], preferred_element_type=f32)
         + jnp.dot(xs1[...] * ic0, wl1[H:2 * H, :], preferred_element_type=f32)
         + jnp.dot(xs2[...] * ic1, wl1[2 * H:3 * H, :], preferred_element_type=f32)
         + jnp.dot(xs3[...] * ic1, wl1[3 * H:4 * H, :], preferred_element_type=f32))
    z = jnp.maximum(z + bl1[...], 0.0)
    z = jnp.maximum(jnp.dot(z, wf1[...], preferred_element_type=f32) + bf1[...], 0.0)
    z = z * (g1[...] * _BN_SCALE) + be1[...]
    z = jnp.maximum(jnp.dot(z, wf2[...], preferred_element_type=f32) + bf2[...], 0.0)
    z = z * (g2[...] * _BN_SCALE) + be2[...]
    logits = jnp.dot(z, wl2[...], preferred_element_type=f32) + bl2[...]
    m = jnp.max(logits, axis=1, keepdims=True)
    s = logits - m
    lse = jnp.log(jnp.sum(jnp.exp(s), axis=1, keepdims=True))
    out_ref[...] = s - lse


def _head(xs, c0, c1, p):
    args = (xs[0], xs[1], xs[2], xs[3], c0, c1,
            p["lin1_W"], p["lin1_b"][None, :],
            p["fc1_W"], p["fc1_b"][None, :], p["bn1_g"][None, :], p["bn1_b"][None, :],
            p["fc2_W"], p["fc2_b"][None, :], p["bn2_g"][None, :], p["bn2_b"][None, :],
            p["lin2_W"], p["lin2_b"][None, :])
    return pl.pallas_call(
        _head_body,
        out_shape=jax.ShapeDtypeStruct((G, 10), f32),
    )(*args)


def kernel(x, params, edge_index, batch):
    p = params
    src = edge_index[0]
    dst = edge_index[1]
    x_p = jnp.pad(x, ((0, NP - N), (0, 0)))
    batch_p = jnp.pad(batch, (0, NP - N), constant_values=G)
    batch_f = batch_p.reshape(NP // 128, 1, 128).astype(f32)

    counts_f = _counts(batch_f)
    counts = counts_f[:, 0].astype(i32)
    k_ = (4 * counts + 4) // 5
    ptr = jnp.concatenate([jnp.zeros((1,), i32), jnp.cumsum(counts)])
    ptr_new = jnp.concatenate([jnp.zeros((1,), i32), jnp.cumsum(k_)])

    lo0c = ptr[:G].astype(f32)[:, None]
    hi0c = ptr[1:G + 1].astype(f32)[:, None]
    lo0r = lo0c.T
    hi0r = hi0c.T
    loNc = ptr_new[:G].astype(f32)[:, None]
    hiNc = (ptr_new[:G] + k_).astype(f32)[:, None]
    loNr = loNc.T
    hiNr = hiNc.T
    ktab = k_.astype(f32)[:, None]
    pntab = ptr_new[:G].astype(f32)[:, None]

    zz = jnp.zeros((RPT, H), f32)

    pa = _agg(x_p, src, dst, zz)
    h0, xs0 = _mlp(x_p, pa[0], pa[1], p["conv0_W1"], p["conv0_b1"],
                   p["conv0_W2"], p["conv0_b2"], p["conv0_g"], p["conv0_be"],
                   lo0r, hi0r)
    pb = _agg(h0, src, dst, zz)
    h1, xs1 = _mlp(h0, pb[0], pb[1], p["conv1_W1"], p["conv1_b1"],
                   p["conv1_W2"], p["conv1_b2"], p["conv1_g"], p["conv1_be"],
                   lo0r, hi0r)

    npf, h1s = _rank(batch_p, ptr, h1, p["pool0_w"][:, None],
                     lo0c, hi0c, lo0r, hi0r, ktab, pntab)
    npos = npf.reshape(NP).astype(i32)

    xnew, nsrc, ndst = _pool(h1s, npos, src, dst)

    pc = _agg(xnew, nsrc, ndst, zz)
    h2, xs2 = _mlp(xnew, pc[0], pc[1], p["conv2_W1"], p["conv2_b1"],
                   p["conv2_W2"], p["conv2_b2"], p["conv2_g"], p["conv2_be"],
                   loNr, hiNr)
    pd = _agg(h2, nsrc, ndst, zz)
    h3, xs3 = _mlp(h2, pd[0], pd[1], p["conv3_W1"], p["conv3_b1"],
                   p["conv3_W2"], p["conv3_b2"], p["conv3_g"], p["conv3_be"],
                   loNr, hiNr)

    return _head((xs0, xs1, xs2, xs3), counts_f, ktab, p)


# speedup vs baseline: 1.0212x; 1.0212x over previous
import functools

import jax
import jax.numpy as jnp
import numpy as np
from jax import lax
from jax.experimental import pallas as pl
from jax.experimental.pallas import tpu as pltpu
from jax.experimental.pallas import tpu_sc as plsc

N = 10000
NP = 10240
E = 320000
G = 64
H = 128
SENT = 10000
BN_EPS = 1e-5
_BN_SCALE = float(1.0 / np.sqrt(1.0 + BN_EPS))

NC, NS = 2, 16
NW = NC * NS
ECH = 40
EPW = E // NW
NCHK = EPW // ECH
RPW = NP // NW
RPT = NP // NS

@functools.cache
def _sc_mesh():
    return plsc.VectorSubcoreMesh(
        core_axis_name="c", subcore_axis_name="s",
        num_cores=NC, num_subcores=NS)

f32 = jnp.float32
i32 = jnp.int32


_NBUF = 8
_NGRP = NCHK // _NBUF
_NTAIL = NCHK - _NGRP * _NBUF


def _agg_body(x_hbm, src_hbm, dst_hbm, zz_hbm, out_hbm, *refs):
    src2 = refs[0]
    dst2 = refs[1]
    rows_v = refs[2:2 + _NBUF]
    acc_sh = refs[2 + _NBUF]
    sems = refs[3 + _NBUF:3 + 2 * _NBUF]
    ssems = refs[3 + 2 * _NBUF:]
    cid = lax.axis_index("c")
    sid = lax.axis_index("s")
    pltpu.sync_copy(zz_hbm, acc_sh.at[pl.ds(sid * RPT, RPT)])
    plsc.subcore_barrier()

    wid = cid * NS + sid

    def start(b):
        return pltpu.async_copy(x_hbm.at[src2.at[b]], rows_v[b], sems[b])

    def scat(b):
        return pltpu.async_copy(rows_v[b], acc_sh.at[dst2.at[b]], ssems[b],
                                add=True)

    @pl.loop(0, _NGRP)
    def _grp(g):
        pltpu.sync_copy(src_hbm.at[wid, pl.ds(g * _NBUF, _NBUF)], src2)
        pltpu.sync_copy(dst_hbm.at[wid, pl.ds(g * _NBUF, _NBUF)], dst2)
        ds = [start(b) for b in range(_NBUF)]
        ws = []
        for b in range(_NBUF):
            ds[b].wait()
            ws.append(scat(b))
        for w in ws:
            w.wait()

    pltpu.sync_copy(src_hbm.at[wid, pl.ds(_NGRP * _NBUF, _NTAIL)],
                    src2.at[pl.ds(0, _NTAIL)])
    pltpu.sync_copy(dst_hbm.at[wid, pl.ds(_NGRP * _NBUF, _NTAIL)],
                    dst2.at[pl.ds(0, _NTAIL)])
    tail = [start(b) for b in range(_NTAIL)]
    wt = []
    for b in range(_NTAIL):
        tail[b].wait()
        wt.append(scat(b))
    for w in wt:
        w.wait()

    plsc.subcore_barrier()
    pltpu.sync_copy(acc_sh.at[pl.ds(sid * RPT, RPT)],
                    out_hbm.at[cid, pl.ds(sid * RPT, RPT)])


@functools.cache
def _agg_kernel():
    return pl.kernel(
        _agg_body,
        out_type=jax.ShapeDtypeStruct((NC, NP, H), f32),
        mesh=_sc_mesh(),
        scratch_types=(
            [pltpu.VMEM((_NBUF, ECH), i32) for _ in range(2)]
            + [pltpu.VMEM((ECH, H), f32) for _ in range(_NBUF)]
            + [pltpu.VMEM_SHARED((NP, H), f32)]
            + [pltpu.SemaphoreType.DMA for _ in range(2 * _NBUF)]
        ),
    )


def _agg(x_p, src, dst, zz):
    return _agg_kernel()(x_p, src.reshape(NW, NCHK, ECH),
                         dst.reshape(NW, NCHK, ECH), zz)


_ECH2 = 400


def _pool_body(h1s_hbm, np_hbm, src_hbm, dst_hbm,
               xnew_hbm, ns_hbm, nd_hbm,
               npos_v, rows_v, npc_v, sidx_v, didx_v, nsb_v, ndb_v):
    cid = lax.axis_index("c")
    sid = lax.axis_index("s")
    wid = cid * NS + sid

    nb = wid * RPW

    @pl.loop(0, RPW // ECH)
    def _nchunk(c):
        pltpu.sync_copy(h1s_hbm.at[pl.ds(nb + c * ECH, ECH)], rows_v)
        pltpu.sync_copy(np_hbm.at[pl.ds(nb + c * ECH, ECH)], npc_v)
        pltpu.sync_copy(rows_v, xnew_hbm.at[npc_v])

    pltpu.sync_copy(np_hbm, npos_v)
    eb = wid * EPW

    @pl.loop(0, EPW // _ECH2)
    def _echunk(c):
        pltpu.sync_copy(src_hbm.at[pl.ds(eb + c * _ECH2, _ECH2)], sidx_v)
        pltpu.sync_copy(dst_hbm.at[pl.ds(eb + c * _ECH2, _ECH2)], didx_v)

        @pl.loop(0, _ECH2 // 16)
        def _vec(k):
            s16 = sidx_v[pl.ds(k * 16, 16)]
            d16 = didx_v[pl.ds(k * 16, 16)]
            ps = plsc.load_gather(npos_v, [s16])
            pd = plsc.load_gather(npos_v, [d16])
            valid = (ps < SENT) & (pd < SENT)
            nsb_v[pl.ds(k * 16, 16)] = jnp.where(valid, ps, SENT + (s16 & 127))
            ndb_v[pl.ds(k * 16, 16)] = jnp.where(valid, pd, SENT + (d16 & 127))

        pltpu.sync_copy(nsb_v, ns_hbm.at[pl.ds(eb + c * _ECH2, _ECH2)])
        pltpu.sync_copy(ndb_v, nd_hbm.at[pl.ds(eb + c * _ECH2, _ECH2)])


@functools.cache
def _pool_kernel():
    return pl.kernel(
        _pool_body,
        out_type=(
            jax.ShapeDtypeStruct((NP, H), f32),
            jax.ShapeDtypeStruct((E,), i32),
            jax.ShapeDtypeStruct((E,), i32),
        ),
        mesh=_sc_mesh(),
        compiler_params=pltpu.CompilerParams(needs_layout_passes=False),
        scratch_types=[
            pltpu.VMEM((NP,), i32),
            pltpu.VMEM((ECH, H), f32),
            pltpu.VMEM((ECH,), i32),
            pltpu.VMEM((_ECH2,), i32),
            pltpu.VMEM((_ECH2,), i32),
            pltpu.VMEM((_ECH2,), i32),
            pltpu.VMEM((_ECH2,), i32),
        ],
    )


def _pool(h1s, npos, src, dst):
    return _pool_kernel()(h1s, npos, src, dst)


def _counts_body(b_ref, out_ref):
    i = pl.program_id(0)

    @pl.when(i == 0)
    def _():
        out_ref[...] = jnp.zeros_like(out_ref)

    b = b_ref[0]
    gcol = lax.broadcasted_iota(i32, (G, 128), 0).astype(f32)
    eq = (jnp.broadcast_to(b, (G, 128)) == gcol).astype(f32)
    out_ref[...] += jnp.sum(eq, axis=1, keepdims=True)


def _counts(batch_f):
    return pl.pallas_call(
        _counts_body,
        grid=(NP // 128,),
        in_specs=[pl.BlockSpec((1, 1, 128), lambda i: (i, 0, 0))],
        out_specs=pl.BlockSpec((G, 1), lambda i: (0, 0)),
        out_shape=jax.ShapeDtypeStruct((G, 1), f32),
    )(batch_f)


R = 1024


def _mlp_body(lo_ref, hi_ref, x_ref, p0_ref, p1_ref, w1_ref, w2_ref, c_ref,
              h_ref, xs_ref):
    i = pl.program_id(0)
    t = x_ref[...] + p0_ref[...] + p1_ref[...]
    a = jnp.dot(t, w1_ref[...], preferred_element_type=f32) + c_ref[0:1, :]
    a = jnp.maximum(a, 0.0)
    a = jnp.dot(a, w2_ref[...], preferred_element_type=f32) + c_ref[1:2, :]
    a = jnp.maximum(a, 0.0)
    h = jnp.maximum(a * (c_ref[2:3, :] * _BN_SCALE) + c_ref[3:4, :], 0.0)
    h_ref[...] = h

    p = lax.broadcasted_iota(i32, (R, G), 0).astype(f32) + (i * R).astype(f32)
    oh = ((p >= lo_ref[...]) & (p < hi_ref[...])).astype(f32)
    vmask = jnp.sum(oh, axis=1, keepdims=True) > 0.0
    hm = jnp.where(vmask, h, 0.0)
    contrib = lax.dot_general(oh, hm, (((0,), (0,)), ((), ())),
                              preferred_element_type=f32)

    @pl.when(i == 0)
    def _():
        xs_ref[...] = jnp.zeros_like(xs_ref)

    xs_ref[...] += contrib


def _mlp(xin, p0, p1, w1, b1, w2, b2, g, be, lo, hi):
    cvec = jnp.stack([b1, b2, g, be], axis=0)
    return pl.pallas_call(
        _mlp_body,
        grid=(NP // R,),
        in_specs=[
            pl.BlockSpec((1, G), lambda i: (0, 0)),
            pl.BlockSpec((1, G), lambda i: (0, 0)),
            pl.BlockSpec((R, H), lambda i: (i, 0)),
            pl.BlockSpec((R, H), lambda i: (i, 0)),
            pl.BlockSpec((R, H), lambda i: (i, 0)),
            pl.BlockSpec((H, H), lambda i: (0, 0)),
            pl.BlockSpec((H, H), lambda i: (0, 0)),
            pl.BlockSpec((4, H), lambda i: (0, 0)),
        ],
        out_specs=[
            pl.BlockSpec((R, H), lambda i: (i, 0)),
            pl.BlockSpec((G, H), lambda i: (0, 0)),
        ],
        out_shape=[
            jax.ShapeDtypeStruct((NP, H), f32),
            jax.ShapeDtypeStruct((G, H), f32),
        ],
    )(lo, hi, xin, p0, p1, w1, w2, cvec)


def _rank_body(batch_s, ptr_s, h_ref, w_ref, loc_ref, hic_ref, lor_ref,
               hir_ref, kt_ref, pnt_ref, np_ref, h1s_ref):
    ic = pl.program_id(0)
    w = w_ref[...]
    nrm = jnp.sqrt(jnp.sum(w * w)) + 1e-16

    hi_blk = h_ref[pl.ds(ic * 128, 128), :]
    s_col = jnp.tanh(jnp.dot(hi_blk, w, preferred_element_type=f32) / nrm)
    h1s_ref[...] = hi_blk * s_col
    eyeb = (lax.broadcasted_iota(i32, (128, 128), 0)
            == lax.broadcasted_iota(i32, (128, 128), 1))
    s_row = jnp.max(jnp.where(eyeb, jnp.broadcast_to(s_col, (128, 128)),
                              -jnp.inf), axis=0, keepdims=True)

    p_i = lax.broadcasted_iota(i32, (G, 128), 1).astype(f32) + (ic * 128).astype(f32)
    ohi = ((p_i >= loc_ref[...]) & (p_i < hic_ref[...])).astype(f32)
    gcol = lax.broadcasted_iota(i32, (G, 128), 0).astype(f32)
    anyi = jnp.sum(ohi, axis=0, keepdims=True)
    bi_row = jnp.sum(ohi * gcol, axis=0, keepdims=True) + (1.0 - anyi) * 64.0
    k_i = jnp.sum(ohi * kt_ref[...], axis=0, keepdims=True)
    pn_i = jnp.sum(ohi * pnt_ref[...], axis=0, keepdims=True)

    idx_i = lax.broadcasted_iota(i32, (128, 128), 1) + ic * 128

    glo = jnp.minimum(batch_s[ic * 128], 63)
    ghi = jnp.minimum(batch_s[jnp.minimum(ic * 128 + 127, N - 1)], 63)
    jlo = ptr_s[glo] // 128
    jhi = (ptr_s[ghi + 1] + 127) // 128

    def jbody(jc, cnt):
        hj = h_ref[pl.ds(jc * 128, 128), :]
        sj = jnp.tanh(jnp.dot(hj, w, preferred_element_type=f32) / nrm)
        p_j = lax.broadcasted_iota(i32, (128, G), 0).astype(f32) + (jc * 128).astype(f32)
        ohj = ((p_j >= lor_ref[...]) & (p_j < hir_ref[...])).astype(f32)
        grow = lax.broadcasted_iota(i32, (128, G), 1).astype(f32)
        anyj = jnp.sum(ohj, axis=1, keepdims=True)
        bj = jnp.sum(ohj * grow, axis=1, keepdims=True) + (1.0 - anyj) * 64.0
        idx_j = lax.broadcasted_iota(i32, (128, 128), 0) + jc * 128
        m = (bj == bi_row) & ((sj > s_row) | ((sj == s_row) & (idx_j < idx_i)))
        return cnt + jnp.sum(m.astype(f32), axis=0, keepdims=True)

    cnt = lax.fori_loop(jlo, jhi, jbody, jnp.zeros((1, 128), f32))

    kept = cnt < k_i
    lane = lax.broadcasted_iota(i32, (1, 128), 1).astype(f32)
    np_ref[0] = jnp.where(kept, pn_i + cnt, jnp.float32(SENT) + lane)


def _rank(batch_p, ptr, h1, w, loc, hic, lor, hir, ktab, pntab):
    grid_spec = pltpu.PrefetchScalarGridSpec(
        num_scalar_prefetch=2,
        grid=(NP // 128,),
        in_specs=[
            pl.BlockSpec((NP, H), lambda ic, *_: (0, 0)),
            pl.BlockSpec((H, 1), lambda ic, *_: (0, 0)),
            pl.BlockSpec((G, 1), lambda ic, *_: (0, 0)),
            pl.BlockSpec((G, 1), lambda ic, *_: (0, 0)),
            pl.BlockSpec((1, G), lambda ic, *_: (0, 0)),
            pl.BlockSpec((1, G), lambda ic, *_: (0, 0)),
            pl.BlockSpec((G, 1), lambda ic, *_: (0, 0)),
            pl.BlockSpec((G, 1), lambda ic, *_: (0, 0)),
        ],
        out_specs=[
            pl.BlockSpec((1, 1, 128), lambda ic, *_: (ic, 0, 0)),
            pl.BlockSpec((128, H), lambda ic, *_: (ic, 0)),
        ],
    )
    return pl.pallas_call(
        _rank_body,
        grid_spec=grid_spec,
        out_shape=[
            jax.ShapeDtypeStruct((NP // 128, 1, 128), f32),
            jax.ShapeDtypeStruct((NP, H), f32),
        ],
    )(batch_p, ptr, h1, w, loc, hic, lor, hir, ktab, pntab)


def _head_body(xs0, xs1, xs2, xs3, c0, c1, wl1, bl1, wf1, bf1, g1, be1,
               wf2, bf2, g2, be2, wl2, bl2, out_ref):
    ic0 = 1.0 / jnp.maximum(c0[...], 1.0)
    ic1 = 1.0 / jnp.maximum(c1[...], 1.0)
    z = (jnp.dot(xs0[...] * ic0, wl1[0:H, :], preferred_element_type=f32)
         + jnp.dot(xs1[...] * ic0, wl1[H:2 * H, :], preferred_element_type=f32)
         + jnp.dot(xs2[...] * ic1, wl1[2 * H:3 * H, :], preferred_element_type=f32)
         + jnp.dot(xs3[...] * ic1, wl1[3 * H:4 * H, :], preferred_element_type=f32))
    z = jnp.maximum(z + bl1[...], 0.0)
    z = jnp.maximum(jnp.dot(z, wf1[...], preferred_element_type=f32) + bf1[...], 0.0)
    z = z * (g1[...] * _BN_SCALE) + be1[...]
    z = jnp.maximum(jnp.dot(z, wf2[...], preferred_element_type=f32) + bf2[...], 0.0)
    z = z * (g2[...] * _BN_SCALE) + be2[...]
    logits = jnp.dot(z, wl2[...], preferred_element_type=f32) + bl2[...]
    m = jnp.max(logits, axis=1, keepdims=True)
    s = logits - m
    lse = jnp.log(jnp.sum(jnp.exp(s), axis=1, keepdims=True))
    out_ref[...] = s - lse


def _head(xs, c0, c1, p):
    args = (xs[0], xs[1], xs[2], xs[3], c0, c1,
            p["lin1_W"], p["lin1_b"][None, :],
            p["fc1_W"], p["fc1_b"][None, :], p["bn1_g"][None, :], p["bn1_b"][None, :],
            p["fc2_W"], p["fc2_b"][None, :], p["bn2_g"][None, :], p["bn2_b"][None, :],
            p["lin2_W"], p["lin2_b"][None, :])
    return pl.pallas_call(
        _head_body,
        out_shape=jax.ShapeDtypeStruct((G, 10), f32),
    )(*args)


def kernel(x, params, edge_index, batch):
    p = params
    src = edge_index[0]
    dst = edge_index[1]
    x_p = jnp.pad(x, ((0, NP - N), (0, 0)))
    batch_p = jnp.pad(batch, (0, NP - N), constant_values=G)
    batch_f = batch_p.reshape(NP // 128, 1, 128).astype(f32)

    counts_f = _counts(batch_f)
    counts = counts_f[:, 0].astype(i32)
    k_ = (4 * counts + 4) // 5
    ptr = jnp.concatenate([jnp.zeros((1,), i32), jnp.cumsum(counts)])
    ptr_new = jnp.concatenate([jnp.zeros((1,), i32), jnp.cumsum(k_)])

    lo0c = ptr[:G].astype(f32)[:, None]
    hi0c = ptr[1:G + 1].astype(f32)[:, None]
    lo0r = lo0c.T
    hi0r = hi0c.T
    loNc = ptr_new[:G].astype(f32)[:, None]
    hiNc = (ptr_new[:G] + k_).astype(f32)[:, None]
    loNr = loNc.T
    hiNr = hiNc.T
    ktab = k_.astype(f32)[:, None]
    pntab = ptr_new[:G].astype(f32)[:, None]

    zz = jnp.zeros((RPT, H), f32)

    pa = _agg(x_p, src, dst, zz)
    h0, xs0 = _mlp(x_p, pa[0], pa[1], p["conv0_W1"], p["conv0_b1"],
                   p["conv0_W2"], p["conv0_b2"], p["conv0_g"], p["conv0_be"],
                   lo0r, hi0r)
    pb = _agg(h0, src, dst, zz)
    h1, xs1 = _mlp(h0, pb[0], pb[1], p["conv1_W1"], p["conv1_b1"],
                   p["conv1_W2"], p["conv1_b2"], p["conv1_g"], p["conv1_be"],
                   lo0r, hi0r)

    npf, h1s = _rank(batch_p, ptr, h1, p["pool0_w"][:, None],
                     lo0c, hi0c, lo0r, hi0r, ktab, pntab)
    npos = npf.reshape(NP).astype(i32)

    xnew, nsrc, ndst = _pool(h1s, npos, src, dst)

    pc = _agg(xnew, nsrc, ndst, zz)
    h2, xs2 = _mlp(xnew, pc[0], pc[1], p["conv2_W1"], p["conv2_b1"],
                   p["conv2_W2"], p["conv2_b2"], p["conv2_g"], p["conv2_be"],
                   loNr, hiNr)
    pd = _agg(h2, nsrc, ndst, zz)
    h3, xs3 = _mlp(h2, pd[0], pd[1], p["conv3_W1"], p["conv3_b1"],
                   p["conv3_W2"], p["conv3_b2"], p["conv3_g"], p["conv3_be"],
                   loNr, hiNr)

    return _head((xs0, xs1, xs2, xs3), counts_f, ktab, p)


# speedup vs baseline: 1.1873x; 1.1626x over previous
import functools

import jax
import jax.numpy as jnp
import numpy as np
from jax import lax
from jax.experimental import pallas as pl
from jax.experimental.pallas import tpu as pltpu
from jax.experimental.pallas import tpu_sc as plsc

N = 10000
NP = 10240
E = 320000
G = 64
H = 128
SENT = 10000
BN_EPS = 1e-5
_BN_SCALE = float(1.0 / np.sqrt(1.0 + BN_EPS))

NC, NS = 2, 16
NW = NC * NS
ECH = 80
EPW = E // NW
NCHK = EPW // ECH
RPW = NP // NW
RPT = NP // NS

@functools.cache
def _sc_mesh():
    return plsc.VectorSubcoreMesh(
        core_axis_name="c", subcore_axis_name="s",
        num_cores=NC, num_subcores=NS)

f32 = jnp.float32
i32 = jnp.int32


_NBUF = 4
_NGRP = NCHK // _NBUF
_NTAIL = NCHK - _NGRP * _NBUF


def _agg_body(x_hbm, src_hbm, dst_hbm, zz_hbm, out_hbm, *refs):
    src_v = refs[0:_NBUF]
    dst_v = refs[_NBUF:2 * _NBUF]
    rows_v = refs[2 * _NBUF:3 * _NBUF]
    acc_sh = refs[3 * _NBUF]
    sems = refs[3 * _NBUF + 1:3 * _NBUF + 1 + _NBUF]
    ssems = refs[3 * _NBUF + 1 + _NBUF:]
    cid = lax.axis_index("c")
    sid = lax.axis_index("s")
    pltpu.sync_copy(zz_hbm, acc_sh.at[pl.ds(sid * RPT, RPT)])
    plsc.subcore_barrier()

    wid = cid * NS + sid
    ebase = wid * EPW

    def start(c, b):
        eb = ebase + c * ECH
        pltpu.sync_copy(src_hbm.at[pl.ds(eb, ECH)], src_v[b])
        return pltpu.async_copy(x_hbm.at[src_v[b]], rows_v[b], sems[b])

    def scat(c, b):
        eb = ebase + c * ECH
        pltpu.sync_copy(dst_hbm.at[pl.ds(eb, ECH)], dst_v[b])
        return pltpu.async_copy(rows_v[b], acc_sh.at[dst_v[b]], ssems[b],
                                add=True)

    @pl.loop(0, _NGRP)
    def _grp(g):
        ds = [start(_NBUF * g + b, b) for b in range(_NBUF)]
        ws = []
        for b in range(_NBUF):
            ds[b].wait()
            ws.append(scat(_NBUF * g + b, b))
        for w in ws:
            w.wait()

    tail = [start(_NGRP * _NBUF + b, b) for b in range(_NTAIL)]
    wt = []
    for b in range(_NTAIL):
        tail[b].wait()
        wt.append(scat(_NGRP * _NBUF + b, b))
    for w in wt:
        w.wait()

    plsc.subcore_barrier()
    pltpu.sync_copy(acc_sh.at[pl.ds(sid * RPT, RPT)],
                    out_hbm.at[cid, pl.ds(sid * RPT, RPT)])


@functools.cache
def _agg_kernel():
    return pl.kernel(
        _agg_body,
        out_type=jax.ShapeDtypeStruct((NC, NP, H), f32),
        mesh=_sc_mesh(),
        scratch_types=(
            [pltpu.VMEM((ECH,), i32) for _ in range(2 * _NBUF)]
            + [pltpu.VMEM((ECH, H), f32) for _ in range(_NBUF)]
            + [pltpu.VMEM_SHARED((NP, H), f32)]
            + [pltpu.SemaphoreType.DMA for _ in range(2 * _NBUF)]
        ),
    )


def _agg(x_p, src, dst, zz):
    return _agg_kernel()(x_p, src, dst, zz)


_ECH2 = 400


def _pool_body(h1s_hbm, np_hbm, src_hbm, dst_hbm,
               xnew_hbm, ns_hbm, nd_hbm,
               npos_v, rows_v, npc_v, sidx_v, didx_v, nsb_v, ndb_v):
    cid = lax.axis_index("c")
    sid = lax.axis_index("s")
    wid = cid * NS + sid

    nb = wid * RPW

    @pl.loop(0, RPW // ECH)
    def _nchunk(c):
        pltpu.sync_copy(h1s_hbm.at[pl.ds(nb + c * ECH, ECH)], rows_v)
        pltpu.sync_copy(np_hbm.at[pl.ds(nb + c * ECH, ECH)], npc_v)
        pltpu.sync_copy(rows_v, xnew_hbm.at[npc_v])

    pltpu.sync_copy(np_hbm, npos_v)
    eb = wid * EPW

    @pl.loop(0, EPW // _ECH2)
    def _echunk(c):
        pltpu.sync_copy(src_hbm.at[pl.ds(eb + c * _ECH2, _ECH2)], sidx_v)
        pltpu.sync_copy(dst_hbm.at[pl.ds(eb + c * _ECH2, _ECH2)], didx_v)

        @pl.loop(0, _ECH2 // 16)
        def _vec(k):
            s16 = sidx_v[pl.ds(k * 16, 16)]
            d16 = didx_v[pl.ds(k * 16, 16)]
            ps = plsc.load_gather(npos_v, [s16])
            pd = plsc.load_gather(npos_v, [d16])
            valid = (ps < SENT) & (pd < SENT)
            nsb_v[pl.ds(k * 16, 16)] = jnp.where(valid, ps, SENT + (s16 & 127))
            ndb_v[pl.ds(k * 16, 16)] = jnp.where(valid, pd, SENT + (d16 & 127))

        pltpu.sync_copy(nsb_v, ns_hbm.at[pl.ds(eb + c * _ECH2, _ECH2)])
        pltpu.sync_copy(ndb_v, nd_hbm.at[pl.ds(eb + c * _ECH2, _ECH2)])


@functools.cache
def _pool_kernel():
    return pl.kernel(
        _pool_body,
        out_type=(
            jax.ShapeDtypeStruct((NP, H), f32),
            jax.ShapeDtypeStruct((E,), i32),
            jax.ShapeDtypeStruct((E,), i32),
        ),
        mesh=_sc_mesh(),
        compiler_params=pltpu.CompilerParams(needs_layout_passes=False),
        scratch_types=[
            pltpu.VMEM((NP,), i32),
            pltpu.VMEM((ECH, H), f32),
            pltpu.VMEM((ECH,), i32),
            pltpu.VMEM((_ECH2,), i32),
            pltpu.VMEM((_ECH2,), i32),
            pltpu.VMEM((_ECH2,), i32),
            pltpu.VMEM((_ECH2,), i32),
        ],
    )


def _pool(h1s, npos, src, dst):
    return _pool_kernel()(h1s, npos, src, dst)


def _counts_body(b_ref, out_ref):
    i = pl.program_id(0)

    @pl.when(i == 0)
    def _():
        out_ref[...] = jnp.zeros_like(out_ref)

    b = b_ref[0]
    gcol = lax.broadcasted_iota(i32, (G, 128), 0).astype(f32)
    eq = (jnp.broadcast_to(b, (G, 128)) == gcol).astype(f32)
    out_ref[...] += jnp.sum(eq, axis=1, keepdims=True)


def _counts(batch_f):
    return pl.pallas_call(
        _counts_body,
        grid=(NP // 128,),
        in_specs=[pl.BlockSpec((1, 1, 128), lambda i: (i, 0, 0))],
        out_specs=pl.BlockSpec((G, 1), lambda i: (0, 0)),
        out_shape=jax.ShapeDtypeStruct((G, 1), f32),
    )(batch_f)


R = 1024


def _mlp_body(lo_ref, hi_ref, x_ref, p0_ref, p1_ref, w1_ref, w2_ref, c_ref,
              h_ref, xs_ref):
    i = pl.program_id(0)
    t = x_ref[...] + p0_ref[...] + p1_ref[...]
    a = jnp.dot(t, w1_ref[...], preferred_element_type=f32) + c_ref[0:1, :]
    a = jnp.maximum(a, 0.0)
    a = jnp.dot(a, w2_ref[...], preferred_element_type=f32) + c_ref[1:2, :]
    a = jnp.maximum(a, 0.0)
    h = jnp.maximum(a * (c_ref[2:3, :] * _BN_SCALE) + c_ref[3:4, :], 0.0)
    h_ref[...] = h

    p = lax.broadcasted_iota(i32, (R, G), 0).astype(f32) + (i * R).astype(f32)
    oh = ((p >= lo_ref[...]) & (p < hi_ref[...])).astype(f32)
    vmask = jnp.sum(oh, axis=1, keepdims=True) > 0.0
    hm = jnp.where(vmask, h, 0.0)
    contrib = lax.dot_general(oh, hm, (((0,), (0,)), ((), ())),
                              preferred_element_type=f32)

    @pl.when(i == 0)
    def _():
        xs_ref[...] = jnp.zeros_like(xs_ref)

    xs_ref[...] += contrib


def _mlp(xin, p0, p1, w1, b1, w2, b2, g, be, lo, hi):
    cvec = jnp.stack([b1, b2, g, be], axis=0)
    return pl.pallas_call(
        _mlp_body,
        grid=(NP // R,),
        in_specs=[
            pl.BlockSpec((1, G), lambda i: (0, 0)),
            pl.BlockSpec((1, G), lambda i: (0, 0)),
            pl.BlockSpec((R, H), lambda i: (i, 0)),
            pl.BlockSpec((R, H), lambda i: (i, 0)),
            pl.BlockSpec((R, H), lambda i: (i, 0)),
            pl.BlockSpec((H, H), lambda i: (0, 0)),
            pl.BlockSpec((H, H), lambda i: (0, 0)),
            pl.BlockSpec((4, H), lambda i: (0, 0)),
        ],
        out_specs=[
            pl.BlockSpec((R, H), lambda i: (i, 0)),
            pl.BlockSpec((G, H), lambda i: (0, 0)),
        ],
        out_shape=[
            jax.ShapeDtypeStruct((NP, H), f32),
            jax.ShapeDtypeStruct((G, H), f32),
        ],
    )(lo, hi, xin, p0, p1, w1, w2, cvec)


def _rank_body(batch_s, ptr_s, h_ref, w_ref, loc_ref, hic_ref, lor_ref,
               hir_ref, kt_ref, pnt_ref, np_ref, h1s_ref):
    ic = pl.program_id(0)
    w = w_ref[...]
    nrm = jnp.sqrt(jnp.sum(w * w)) + 1e-16

    hi_blk = h_ref[pl.ds(ic * 128, 128), :]
    s_col = jnp.tanh(jnp.dot(hi_blk, w, preferred_element_type=f32) / nrm)
    h1s_ref[...] = hi_blk * s_col
    eyeb = (lax.broadcasted_iota(i32, (128, 128), 0)
            == lax.broadcasted_iota(i32, (128, 128), 1))
    s_row = jnp.max(jnp.where(eyeb, jnp.broadcast_to(s_col, (128, 128)),
                              -jnp.inf), axis=0, keepdims=True)

    p_i = lax.broadcasted_iota(i32, (G, 128), 1).astype(f32) + (ic * 128).astype(f32)
    ohi = ((p_i >= loc_ref[...]) & (p_i < hic_ref[...])).astype(f32)
    gcol = lax.broadcasted_iota(i32, (G, 128), 0).astype(f32)
    anyi = jnp.sum(ohi, axis=0, keepdims=True)
    bi_row = jnp.sum(ohi * gcol, axis=0, keepdims=True) + (1.0 - anyi) * 64.0
    k_i = jnp.sum(ohi * kt_ref[...], axis=0, keepdims=True)
    pn_i = jnp.sum(ohi * pnt_ref[...], axis=0, keepdims=True)

    idx_i = lax.broadcasted_iota(i32, (128, 128), 1) + ic * 128

    glo = jnp.minimum(batch_s[ic * 128], 63)
    ghi = jnp.minimum(batch_s[jnp.minimum(ic * 128 + 127, N - 1)], 63)
    jlo = ptr_s[glo] // 128
    jhi = (ptr_s[ghi + 1] + 127) // 128

    def jbody(jc, cnt):
        hj = h_ref[pl.ds(jc * 128, 128), :]
        sj = jnp.tanh(jnp.dot(hj, w, preferred_element_type=f32) / nrm)
        p_j = lax.broadcasted_iota(i32, (128, G), 0).astype(f32) + (jc * 128).astype(f32)
        ohj = ((p_j >= lor_ref[...]) & (p_j < hir_ref[...])).astype(f32)
        grow = lax.broadcasted_iota(i32, (128, G), 1).astype(f32)
        anyj = jnp.sum(ohj, axis=1, keepdims=True)
        bj = jnp.sum(ohj * grow, axis=1, keepdims=True) + (1.0 - anyj) * 64.0
        idx_j = lax.broadcasted_iota(i32, (128, 128), 0) + jc * 128
        m = (bj == bi_row) & ((sj > s_row) | ((sj == s_row) & (idx_j < idx_i)))
        return cnt + jnp.sum(m.astype(f32), axis=0, keepdims=True)

    cnt = lax.fori_loop(jlo, jhi, jbody, jnp.zeros((1, 128), f32))

    kept = cnt < k_i
    lane = lax.broadcasted_iota(i32, (1, 128), 1).astype(f32)
    np_ref[0] = jnp.where(kept, pn_i + cnt, jnp.float32(SENT) + lane)


def _rank(batch_p, ptr, h1, w, loc, hic, lor, hir, ktab, pntab):
    grid_spec = pltpu.PrefetchScalarGridSpec(
        num_scalar_prefetch=2,
        grid=(NP // 128,),
        in_specs=[
            pl.BlockSpec((NP, H), lambda ic, *_: (0, 0)),
            pl.BlockSpec((H, 1), lambda ic, *_: (0, 0)),
            pl.BlockSpec((G, 1), lambda ic, *_: (0, 0)),
            pl.BlockSpec((G, 1), lambda ic, *_: (0, 0)),
            pl.BlockSpec((1, G), lambda ic, *_: (0, 0)),
            pl.BlockSpec((1, G), lambda ic, *_: (0, 0)),
            pl.BlockSpec((G, 1), lambda ic, *_: (0, 0)),
            pl.BlockSpec((G, 1), lambda ic, *_: (0, 0)),
        ],
        out_specs=[
            pl.BlockSpec((1, 1, 128), lambda ic, *_: (ic, 0, 0)),
            pl.BlockSpec((128, H), lambda ic, *_: (ic, 0)),
        ],
    )
    return pl.pallas_call(
        _rank_body,
        grid_spec=grid_spec,
        out_shape=[
            jax.ShapeDtypeStruct((NP // 128, 1, 128), f32),
            jax.ShapeDtypeStruct((NP, H), f32),
        ],
    )(batch_p, ptr, h1, w, loc, hic, lor, hir, ktab, pntab)


def _head_body(xs0, xs1, xs2, xs3, c0, c1, wl1, bl1, wf1, bf1, g1, be1,
               wf2, bf2, g2, be2, wl2, bl2, out_ref):
    ic0 = 1.0 / jnp.maximum(c0[...], 1.0)
    ic1 = 1.0 / jnp.maximum(c1[...], 1.0)
    z = (jnp.dot(xs0[...] * ic0, wl1[0:H, :], preferred_element_type=f32)
         + jnp.dot(xs1[...] * ic0, wl1[H:2 * H, :], preferred_element_type=f32)
         + jnp.dot(xs2[...] * ic1, wl1[2 * H:3 * H, :], preferred_element_type=f32)
         + jnp.dot(xs3[...] * ic1, wl1[3 * H:4 * H, :], preferred_element_type=f32))
    z = jnp.maximum(z + bl1[...], 0.0)
    z = jnp.maximum(jnp.dot(z, wf1[...], preferred_element_type=f32) + bf1[...], 0.0)
    z = z * (g1[...] * _BN_SCALE) + be1[...]
    z = jnp.maximum(jnp.dot(z, wf2[...], preferred_element_type=f32) + bf2[...], 0.0)
    z = z * (g2[...] * _BN_SCALE) + be2[...]
    logits = jnp.dot(z, wl2[...], preferred_element_type=f32) + bl2[...]
    m = jnp.max(logits, axis=1, keepdims=True)
    s = logits - m
    lse = jnp.log(jnp.sum(jnp.exp(s), axis=1, keepdims=True))
    out_ref[...] = s - lse


def _head(xs, c0, c1, p):
    args = (xs[0], xs[1], xs[2], xs[3], c0, c1,
            p["lin1_W"], p["lin1_b"][None, :],
            p["fc1_W"], p["fc1_b"][None, :], p["bn1_g"][None, :], p["bn1_b"][None, :],
            p["fc2_W"], p["fc2_b"][None, :], p["bn2_g"][None, :], p["bn2_b"][None, :],
            p["lin2_W"], p["lin2_b"][None, :])
    return pl.pallas_call(
        _head_body,
        out_shape=jax.ShapeDtypeStruct((G, 10), f32),
    )(*args)


def kernel(x, params, edge_index, batch):
    p = params
    src = edge_index[0]
    dst = edge_index[1]
    x_p = jnp.pad(x, ((0, NP - N), (0, 0)))
    batch_p = jnp.pad(batch, (0, NP - N), constant_values=G)
    batch_f = batch_p.reshape(NP // 128, 1, 128).astype(f32)

    counts_f = _counts(batch_f)
    counts = counts_f[:, 0].astype(i32)
    k_ = (4 * counts + 4) // 5
    ptr = jnp.concatenate([jnp.zeros((1,), i32), jnp.cumsum(counts)])
    ptr_new = jnp.concatenate([jnp.zeros((1,), i32), jnp.cumsum(k_)])

    lo0c = ptr[:G].astype(f32)[:, None]
    hi0c = ptr[1:G + 1].astype(f32)[:, None]
    lo0r = lo0c.T
    hi0r = hi0c.T
    loNc = ptr_new[:G].astype(f32)[:, None]
    hiNc = (ptr_new[:G] + k_).astype(f32)[:, None]
    loNr = loNc.T
    hiNr = hiNc.T
    ktab = k_.astype(f32)[:, None]
    pntab = ptr_new[:G].astype(f32)[:, None]

    zz = jnp.zeros((RPT, H), f32)

    pa = _agg(x_p, src, dst, zz)
    h0, xs0 = _mlp(x_p, pa[0], pa[1], p["conv0_W1"], p["conv0_b1"],
                   p["conv0_W2"], p["conv0_b2"], p["conv0_g"], p["conv0_be"],
                   lo0r, hi0r)
    pb = _agg(h0, src, dst, zz)
    h1, xs1 = _mlp(h0, pb[0], pb[1], p["conv1_W1"], p["conv1_b1"],
                   p["conv1_W2"], p["conv1_b2"], p["conv1_g"], p["conv1_be"],
                   lo0r, hi0r)

    npf, h1s = _rank(batch_p, ptr, h1, p["pool0_w"][:, None],
                     lo0c, hi0c, lo0r, hi0r, ktab, pntab)
    npos = npf.reshape(NP).astype(i32)

    xnew, nsrc, ndst = _pool(h1s, npos, src, dst)

    pc = _agg(xnew, nsrc, ndst, zz)
    h2, xs2 = _mlp(xnew, pc[0], pc[1], p["conv2_W1"], p["conv2_b1"],
                   p["conv2_W2"], p["conv2_b2"], p["conv2_g"], p["conv2_be"],
                   loNr, hiNr)
    pd = _agg(h2, nsrc, ndst, zz)
    h3, xs3 = _mlp(h2, pd[0], pd[1], p["conv3_W1"], p["conv3_b1"],
                   p["conv3_W2"], p["conv3_b2"], p["conv3_g"], p["conv3_be"],
                   loNr, hiNr)

    return _head((xs0, xs1, xs2, xs3), counts_f, ktab, p)


# speedup vs baseline: 1.2205x; 1.0280x over previous
import functools

import jax
import jax.numpy as jnp
import numpy as np
from jax import lax
from jax.experimental import pallas as pl
from jax.experimental.pallas import tpu as pltpu
from jax.experimental.pallas import tpu_sc as plsc

N = 10000
NP = 10240
E = 320000
G = 64
H = 128
SENT = 10000
BN_EPS = 1e-5
_BN_SCALE = float(1.0 / np.sqrt(1.0 + BN_EPS))

NC, NS = 2, 16
NW = NC * NS
ECH = 80
EPW = E // NW
NCHK = EPW // ECH
RPW = NP // NW
RPT = NP // NS

@functools.cache
def _sc_mesh():
    return plsc.VectorSubcoreMesh(
        core_axis_name="c", subcore_axis_name="s",
        num_cores=NC, num_subcores=NS)

f32 = jnp.float32
i32 = jnp.int32


_NBUF = 4
_NGRP = NCHK // _NBUF
_NTAIL = NCHK - _NGRP * _NBUF


def _agg_body(x_hbm, src_hbm, dst_hbm, zz_hbm, out_hbm, *refs):
    src_v = refs[0:_NBUF]
    dst_v = refs[_NBUF:2 * _NBUF]
    rows_v = refs[2 * _NBUF:3 * _NBUF]
    acc_sh = refs[3 * _NBUF]
    sems = refs[3 * _NBUF + 1:3 * _NBUF + 1 + _NBUF]
    ssems = refs[3 * _NBUF + 1 + _NBUF:]
    cid = lax.axis_index("c")
    sid = lax.axis_index("s")
    pltpu.sync_copy(zz_hbm, acc_sh.at[pl.ds(sid * RPT, RPT)])
    plsc.subcore_barrier()

    wid = cid * NS + sid
    ebase = wid * EPW

    def start(c, b):
        eb = ebase + c * ECH
        pltpu.sync_copy(src_hbm.at[pl.ds(eb, ECH)], src_v[b])
        return pltpu.async_copy(x_hbm.at[src_v[b]], rows_v[b], sems[b])

    def scat(c, b):
        eb = ebase + c * ECH
        pltpu.sync_copy(dst_hbm.at[pl.ds(eb, ECH)], dst_v[b])
        return pltpu.async_copy(rows_v[b], acc_sh.at[dst_v[b]], ssems[b],
                                add=True)

    @pl.loop(0, _NGRP)
    def _grp(g):
        ds = [start(_NBUF * g + b, b) for b in range(_NBUF)]
        ws = []
        for b in range(_NBUF):
            ds[b].wait()
            ws.append(scat(_NBUF * g + b, b))
        for w in ws:
            w.wait()

    tail = [start(_NGRP * _NBUF + b, b) for b in range(_NTAIL)]
    wt = []
    for b in range(_NTAIL):
        tail[b].wait()
        wt.append(scat(_NGRP * _NBUF + b, b))
    for w in wt:
        w.wait()

    plsc.subcore_barrier()
    pltpu.sync_copy(acc_sh.at[pl.ds(sid * RPT, RPT)],
                    out_hbm.at[cid, pl.ds(sid * RPT, RPT)])


@functools.cache
def _agg_kernel():
    return pl.kernel(
        _agg_body,
        out_type=jax.ShapeDtypeStruct((NC, NP, H), f32),
        mesh=_sc_mesh(),
        scratch_types=(
            [pltpu.VMEM((ECH,), i32) for _ in range(2 * _NBUF)]
            + [pltpu.VMEM((ECH, H), f32) for _ in range(_NBUF)]
            + [pltpu.VMEM_SHARED((NP, H), f32)]
            + [pltpu.SemaphoreType.DMA for _ in range(2 * _NBUF)]
        ),
    )


def _agg(x_p, src, dst, zz):
    return _agg_kernel()(x_p, src, dst, zz)


_ECH2 = 2000
_NCH = RPW // ECH


def _pool_body(h1s_hbm, np_hbm, src_hbm, dst_hbm,
               xnew_hbm, ns_hbm, nd_hbm, *refs):
    rows_v = refs[0:_NCH]
    npc_v = refs[_NCH:2 * _NCH]
    gsems = refs[2 * _NCH:3 * _NCH]
    wsems = refs[3 * _NCH:4 * _NCH]
    npos_v, sidx_v, didx_v, nsb_v, ndb_v = refs[4 * _NCH:]
    cid = lax.axis_index("c")
    sid = lax.axis_index("s")
    wid = cid * NS + sid

    nb = wid * RPW
    dh = []
    for b in range(_NCH):
        pltpu.sync_copy(np_hbm.at[pl.ds(nb + b * ECH, ECH)], npc_v[b])
        dh.append(pltpu.async_copy(h1s_hbm.at[pl.ds(nb + b * ECH, ECH)],
                                   rows_v[b], gsems[b]))
    ws = []
    for b in range(_NCH):
        dh[b].wait()
        ws.append(pltpu.async_copy(rows_v[b], xnew_hbm.at[npc_v[b]],
                                   wsems[b]))
    for w in ws:
        w.wait()

    pltpu.sync_copy(np_hbm, npos_v)
    eb = wid * EPW

    @pl.loop(0, EPW // _ECH2)
    def _echunk(c):
        pltpu.sync_copy(src_hbm.at[pl.ds(eb + c * _ECH2, _ECH2)], sidx_v)
        pltpu.sync_copy(dst_hbm.at[pl.ds(eb + c * _ECH2, _ECH2)], didx_v)

        @pl.loop(0, _ECH2 // 16)
        def _vec(k):
            s16 = sidx_v[pl.ds(k * 16, 16)]
            d16 = didx_v[pl.ds(k * 16, 16)]
            ps = plsc.load_gather(npos_v, [s16])
            pd = plsc.load_gather(npos_v, [d16])
            valid = (ps < SENT) & (pd < SENT)
            nsb_v[pl.ds(k * 16, 16)] = jnp.where(valid, ps, SENT + (s16 & 127))
            ndb_v[pl.ds(k * 16, 16)] = jnp.where(valid, pd, SENT + (d16 & 127))

        pltpu.sync_copy(nsb_v, ns_hbm.at[pl.ds(eb + c * _ECH2, _ECH2)])
        pltpu.sync_copy(ndb_v, nd_hbm.at[pl.ds(eb + c * _ECH2, _ECH2)])


@functools.cache
def _pool_kernel():
    return pl.kernel(
        _pool_body,
        out_type=(
            jax.ShapeDtypeStruct((NP, H), f32),
            jax.ShapeDtypeStruct((E,), i32),
            jax.ShapeDtypeStruct((E,), i32),
        ),
        mesh=_sc_mesh(),
        compiler_params=pltpu.CompilerParams(needs_layout_passes=False),
        scratch_types=(
            [pltpu.VMEM((ECH, H), f32) for _ in range(_NCH)]
            + [pltpu.VMEM((ECH,), i32) for _ in range(_NCH)]
            + [pltpu.SemaphoreType.DMA for _ in range(2 * _NCH)]
            + [
                pltpu.VMEM((NP,), i32),
                pltpu.VMEM((_ECH2,), i32),
                pltpu.VMEM((_ECH2,), i32),
                pltpu.VMEM((_ECH2,), i32),
                pltpu.VMEM((_ECH2,), i32),
            ]
        ),
    )


def _pool(h1s, npos, src, dst):
    return _pool_kernel()(h1s, npos, src, dst)


def _counts_body(b_ref, out_ref):
    i = pl.program_id(0)

    @pl.when(i == 0)
    def _():
        out_ref[...] = jnp.zeros_like(out_ref)

    b = b_ref[0]
    gcol = lax.broadcasted_iota(i32, (G, 128), 0).astype(f32)
    eq = (jnp.broadcast_to(b, (G, 128)) == gcol).astype(f32)
    out_ref[...] += jnp.sum(eq, axis=1, keepdims=True)


def _counts(batch_f):
    return pl.pallas_call(
        _counts_body,
        grid=(NP // 128,),
        in_specs=[pl.BlockSpec((1, 1, 128), lambda i: (i, 0, 0))],
        out_specs=pl.BlockSpec((G, 1), lambda i: (0, 0)),
        out_shape=jax.ShapeDtypeStruct((G, 1), f32),
    )(batch_f)


R = 1024


def _mlp_body(lo_ref, hi_ref, x_ref, p0_ref, p1_ref, w1_ref, w2_ref, c_ref,
              h_ref, xs_ref):
    i = pl.program_id(0)
    t = x_ref[...] + p0_ref[...] + p1_ref[...]
    a = jnp.dot(t, w1_ref[...], preferred_element_type=f32) + c_ref[0:1, :]
    a = jnp.maximum(a, 0.0)
    a = jnp.dot(a, w2_ref[...], preferred_element_type=f32) + c_ref[1:2, :]
    a = jnp.maximum(a, 0.0)
    h = jnp.maximum(a * (c_ref[2:3, :] * _BN_SCALE) + c_ref[3:4, :], 0.0)
    h_ref[...] = h

    p = lax.broadcasted_iota(i32, (R, G), 0).astype(f32) + (i * R).astype(f32)
    oh = ((p >= lo_ref[...]) & (p < hi_ref[...])).astype(f32)
    vmask = jnp.sum(oh, axis=1, keepdims=True) > 0.0
    hm = jnp.where(vmask, h, 0.0)
    contrib = lax.dot_general(oh, hm, (((0,), (0,)), ((), ())),
                              preferred_element_type=f32)

    @pl.when(i == 0)
    def _():
        xs_ref[...] = jnp.zeros_like(xs_ref)

    xs_ref[...] += contrib


def _mlp(xin, p0, p1, w1, b1, w2, b2, g, be, lo, hi):
    cvec = jnp.stack([b1, b2, g, be], axis=0)
    return pl.pallas_call(
        _mlp_body,
        grid=(NP // R,),
        in_specs=[
            pl.BlockSpec((1, G), lambda i: (0, 0)),
            pl.BlockSpec((1, G), lambda i: (0, 0)),
            pl.BlockSpec((R, H), lambda i: (i, 0)),
            pl.BlockSpec((R, H), lambda i: (i, 0)),
            pl.BlockSpec((R, H), lambda i: (i, 0)),
            pl.BlockSpec((H, H), lambda i: (0, 0)),
            pl.BlockSpec((H, H), lambda i: (0, 0)),
            pl.BlockSpec((4, H), lambda i: (0, 0)),
        ],
        out_specs=[
            pl.BlockSpec((R, H), lambda i: (i, 0)),
            pl.BlockSpec((G, H), lambda i: (0, 0)),
        ],
        out_shape=[
            jax.ShapeDtypeStruct((NP, H), f32),
            jax.ShapeDtypeStruct((G, H), f32),
        ],
    )(lo, hi, xin, p0, p1, w1, w2, cvec)


def _rank_body(batch_s, ptr_s, h_ref, w_ref, loc_ref, hic_ref, lor_ref,
               hir_ref, kt_ref, pnt_ref, np_ref, h1s_ref):
    ic = pl.program_id(0)
    w = w_ref[...]
    nrm = jnp.sqrt(jnp.sum(w * w)) + 1e-16

    hi_blk = h_ref[pl.ds(ic * 128, 128), :]
    s_col = jnp.tanh(jnp.dot(hi_blk, w, preferred_element_type=f32) / nrm)
    h1s_ref[...] = hi_blk * s_col
    eyeb = (lax.broadcasted_iota(i32, (128, 128), 0)
            == lax.broadcasted_iota(i32, (128, 128), 1))
    s_row = jnp.max(jnp.where(eyeb, jnp.broadcast_to(s_col, (128, 128)),
                              -jnp.inf), axis=0, keepdims=True)

    p_i = lax.broadcasted_iota(i32, (G, 128), 1).astype(f32) + (ic * 128).astype(f32)
    ohi = ((p_i >= loc_ref[...]) & (p_i < hic_ref[...])).astype(f32)
    gcol = lax.broadcasted_iota(i32, (G, 128), 0).astype(f32)
    anyi = jnp.sum(ohi, axis=0, keepdims=True)
    bi_row = jnp.sum(ohi * gcol, axis=0, keepdims=True) + (1.0 - anyi) * 64.0
    k_i = jnp.sum(ohi * kt_ref[...], axis=0, keepdims=True)
    pn_i = jnp.sum(ohi * pnt_ref[...], axis=0, keepdims=True)

    idx_i = lax.broadcasted_iota(i32, (128, 128), 1) + ic * 128

    glo = jnp.minimum(batch_s[ic * 128], 63)
    ghi = jnp.minimum(batch_s[jnp.minimum(ic * 128 + 127, N - 1)], 63)
    jlo = ptr_s[glo] // 128
    jhi = (ptr_s[ghi + 1] + 127) // 128

    def jbody(jc, cnt):
        hj = h_ref[pl.ds(jc * 128, 128), :]
        sj = jnp.tanh(jnp.dot(hj, w, preferred_element_type=f32) / nrm)
        p_j = lax.broadcasted_iota(i32, (128, G), 0).astype(f32) + (jc * 128).astype(f32)
        ohj = ((p_j >= lor_ref[...]) & (p_j < hir_ref[...])).astype(f32)
        grow = lax.broadcasted_iota(i32, (128, G), 1).astype(f32)
        anyj = jnp.sum(ohj, axis=1, keepdims=True)
        bj = jnp.sum(ohj * grow, axis=1, keepdims=True) + (1.0 - anyj) * 64.0
        idx_j = lax.broadcasted_iota(i32, (128, 128), 0) + jc * 128
        m = (bj == bi_row) & ((sj > s_row) | ((sj == s_row) & (idx_j < idx_i)))
        return cnt + jnp.sum(m.astype(f32), axis=0, keepdims=True)

    cnt = lax.fori_loop(jlo, jhi, jbody, jnp.zeros((1, 128), f32))

    kept = cnt < k_i
    lane = lax.broadcasted_iota(i32, (1, 128), 1).astype(f32)
    np_ref[0] = jnp.where(kept, pn_i + cnt, jnp.float32(SENT) + lane)


def _rank(batch_p, ptr, h1, w, loc, hic, lor, hir, ktab, pntab):
    grid_spec = pltpu.PrefetchScalarGridSpec(
        num_scalar_prefetch=2,
        grid=(NP // 128,),
        in_specs=[
            pl.BlockSpec((NP, H), lambda ic, *_: (0, 0)),
            pl.BlockSpec((H, 1), lambda ic, *_: (0, 0)),
            pl.BlockSpec((G, 1), lambda ic, *_: (0, 0)),
            pl.BlockSpec((G, 1), lambda ic, *_: (0, 0)),
            pl.BlockSpec((1, G), lambda ic, *_: (0, 0)),
            pl.BlockSpec((1, G), lambda ic, *_: (0, 0)),
            pl.BlockSpec((G, 1), lambda ic, *_: (0, 0)),
            pl.BlockSpec((G, 1), lambda ic, *_: (0, 0)),
        ],
        out_specs=[
            pl.BlockSpec((1, 1, 128), lambda ic, *_: (ic, 0, 0)),
            pl.BlockSpec((128, H), lambda ic, *_: (ic, 0)),
        ],
    )
    return pl.pallas_call(
        _rank_body,
        grid_spec=grid_spec,
        out_shape=[
            jax.ShapeDtypeStruct((NP // 128, 1, 128), f32),
            jax.ShapeDtypeStruct((NP, H), f32),
        ],
    )(batch_p, ptr, h1, w, loc, hic, lor, hir, ktab, pntab)


def _head_body(xs0, xs1, xs2, xs3, c0, c1, wl1, bl1, wf1, bf1, g1, be1,
               wf2, bf2, g2, be2, wl2, bl2, out_ref):
    ic0 = 1.0 / jnp.maximum(c0[...], 1.0)
    ic1 = 1.0 / jnp.maximum(c1[...], 1.0)
    z = (jnp.dot(xs0[...] * ic0, wl1[0:H, :], preferred_element_type=f32)
         + jnp.dot(xs1[...] * ic0, wl1[H:2 * H, :], preferred_element_type=f32)
         + jnp.dot(xs2[...] * ic1, wl1[2 * H:3 * H, :], preferred_element_type=f32)
         + jnp.dot(xs3[...] * ic1, wl1[3 * H:4 * H, :], preferred_element_type=f32))
    z = jnp.maximum(z + bl1[...], 0.0)
    z = jnp.maximum(jnp.dot(z, wf1[...], preferred_element_type=f32) + bf1[...], 0.0)
    z = z * (g1[...] * _BN_SCALE) + be1[...]
    z = jnp.maximum(jnp.dot(z, wf2[...], preferred_element_type=f32) + bf2[...], 0.0)
    z = z * (g2[...] * _BN_SCALE) + be2[...]
    logits = jnp.dot(z, wl2[...], preferred_element_type=f32) + bl2[...]
    m = jnp.max(logits, axis=1, keepdims=True)
    s = logits - m
    lse = jnp.log(jnp.sum(jnp.exp(s), axis=1, keepdims=True))
    out_ref[...] = s - lse


def _head(xs, c0, c1, p):
    args = (xs[0], xs[1], xs[2], xs[3], c0, c1,
            p["lin1_W"], p["lin1_b"][None, :],
            p["fc1_W"], p["fc1_b"][None, :], p["bn1_g"][None, :], p["bn1_b"][None, :],
            p["fc2_W"], p["fc2_b"][None, :], p["bn2_g"][None, :], p["bn2_b"][None, :],
            p["lin2_W"], p["lin2_b"][None, :])
    return pl.pallas_call(
        _head_body,
        out_shape=jax.ShapeDtypeStruct((G, 10), f32),
    )(*args)


def kernel(x, params, edge_index, batch):
    p = params
    src = edge_index[0]
    dst = edge_index[1]
    x_p = jnp.pad(x, ((0, NP - N), (0, 0)))
    batch_p = jnp.pad(batch, (0, NP - N), constant_values=G)
    batch_f = batch_p.reshape(NP // 128, 1, 128).astype(f32)

    counts_f = _counts(batch_f)
    counts = counts_f[:, 0].astype(i32)
    k_ = (4 * counts + 4) // 5
    ptr = jnp.concatenate([jnp.zeros((1,), i32), jnp.cumsum(counts)])
    ptr_new = jnp.concatenate([jnp.zeros((1,), i32), jnp.cumsum(k_)])

    lo0c = ptr[:G].astype(f32)[:, None]
    hi0c = ptr[1:G + 1].astype(f32)[:, None]
    lo0r = lo0c.T
    hi0r = hi0c.T
    loNc = ptr_new[:G].astype(f32)[:, None]
    hiNc = (ptr_new[:G] + k_).astype(f32)[:, None]
    loNr = loNc.T
    hiNr = hiNc.T
    ktab = k_.astype(f32)[:, None]
    pntab = ptr_new[:G].astype(f32)[:, None]

    zz = jnp.zeros((RPT, H), f32)

    pa = _agg(x_p, src, dst, zz)
    h0, xs0 = _mlp(x_p, pa[0], pa[1], p["conv0_W1"], p["conv0_b1"],
                   p["conv0_W2"], p["conv0_b2"], p["conv0_g"], p["conv0_be"],
                   lo0r, hi0r)
    pb = _agg(h0, src, dst, zz)
    h1, xs1 = _mlp(h0, pb[0], pb[1], p["conv1_W1"], p["conv1_b1"],
                   p["conv1_W2"], p["conv1_b2"], p["conv1_g"], p["conv1_be"],
                   lo0r, hi0r)

    npf, h1s = _rank(batch_p, ptr, h1, p["pool0_w"][:, None],
                     lo0c, hi0c, lo0r, hi0r, ktab, pntab)
    npos = npf.reshape(NP).astype(i32)

    xnew, nsrc, ndst = _pool(h1s, npos, src, dst)

    pc = _agg(xnew, nsrc, ndst, zz)
    h2, xs2 = _mlp(xnew, pc[0], pc[1], p["conv2_W1"], p["conv2_b1"],
                   p["conv2_W2"], p["conv2_b2"], p["conv2_g"], p["conv2_be"],
                   loNr, hiNr)
    pd = _agg(h2, nsrc, ndst, zz)
    h3, xs3 = _mlp(h2, pd[0], pd[1], p["conv3_W1"], p["conv3_b1"],
                   p["conv3_W2"], p["conv3_b2"], p["conv3_g"], p["conv3_be"],
                   loNr, hiNr)

    return _head((xs0, xs1, xs2, xs3), counts_f, ktab, p)
